# baseline (speedup 1.0000x reference)
import functools

import jax
import jax.numpy as jnp
from jax import lax
from jax.experimental import pallas as pl
from jax.experimental.pallas import tpu as pltpu
from jax.experimental.pallas import tpu_sc as plsc

B = 16384
DIM = 128
D2 = 2 * DIM
VIS = 4096
BM = 256
NB = B // BM

NW = 32
RPW = B // NW
CH = 64
NCH = RPW // CH

_sc_mesh = plsc.VectorSubcoreMesh(core_axis_name="c", subcore_axis_name="s")


@functools.lru_cache(maxsize=None)
def _make_sc_prep(bsz):
  rpw = bsz // NW
  nch = rpw // CH

  @functools.partial(
    pl.kernel,
    mesh=_sc_mesh,
    out_type=[
        jax.ShapeDtypeStruct((bsz, 16), jnp.float32),
        jax.ShapeDtypeStruct((bsz, 16), jnp.float32),
        jax.ShapeDtypeStruct((bsz, D2), jnp.float32),
        jax.ShapeDtypeStruct((bsz, D2), jnp.float32),
        jax.ShapeDtypeStruct((bsz, D2), jnp.float32),
    ],
    scratch_types=[
        pltpu.VMEM((rpw,), jnp.int32),
        pltpu.VMEM((rpw,), jnp.int32),
        pltpu.VMEM((rpw,), jnp.int32),
        pltpu.VMEM((CH, DIM), jnp.float32),
        pltpu.VMEM((CH, DIM), jnp.float32),
        pltpu.VMEM((CH, DIM), jnp.float32),
        pltpu.VMEM((CH, DIM), jnp.float32),
        pltpu.VMEM((CH, DIM), jnp.float32),
        pltpu.VMEM((CH, DIM), jnp.float32),
        pltpu.VMEM((CH, D2), jnp.float32),
        pltpu.VMEM((CH, D2), jnp.float32),
        pltpu.VMEM((CH, D2), jnp.float32),
        pltpu.VMEM((CH, 16), jnp.float32),
        pltpu.VMEM((CH, 16), jnp.float32),
        pltpu.SemaphoreType.DMA,
    ],
  )
  def _sc_prep(bh_hbm, bt_hbm, br_hbm,
               entre_hbm, entim_hbm, entemb_hbm,
               relre_hbm, relim_hbm, relemb_hbm,
               c_hbm, stt_hbm, u_hbm, w_hbm, r_hbm,
               bh_v, bt_v, br_v,
               hre_v, him_v, tre_v, tim_v, rre_v, rim_v,
               h_v, t_v, r_v, c_v, stt_v, sem):
    wid = lax.axis_index("s") * 2 + lax.axis_index("c")
    rbase = wid * rpw
    pltpu.sync_copy(bh_hbm.at[pl.ds(rbase, rpw)], bh_v)
    pltpu.sync_copy(bt_hbm.at[pl.ds(rbase, rpw)], bt_v)
    pltpu.sync_copy(br_hbm.at[pl.ds(rbase, rpw)], br_v)

    def chunk_body(ci, carry):
        off = ci * CH
        ih = bh_v.at[pl.ds(off, CH)]
        it = bt_v.at[pl.ds(off, CH)]
        ir = br_v.at[pl.ds(off, CH)]
        cps = [
            pltpu.make_async_copy(entre_hbm.at[ih], hre_v, sem),
            pltpu.make_async_copy(entim_hbm.at[ih], him_v, sem),
            pltpu.make_async_copy(entre_hbm.at[it], tre_v, sem),
            pltpu.make_async_copy(entim_hbm.at[it], tim_v, sem),
            pltpu.make_async_copy(relre_hbm.at[ir], rre_v, sem),
            pltpu.make_async_copy(relim_hbm.at[ir], rim_v, sem),
            pltpu.make_async_copy(entemb_hbm.at[ih], h_v, sem),
            pltpu.make_async_copy(entemb_hbm.at[it], t_v, sem),
            pltpu.make_async_copy(relemb_hbm.at[ir], r_v, sem),
        ]
        for cp in cps:
            cp.start()
        for cp in cps:
            cp.wait()

        def row_body(row, carry2):
                def ck(k, acc):
                    sl = pl.ds(k * 16, 16)
                    hre = hre_v[row, sl]
                    him = him_v[row, sl]
                    tre = tre_v[row, sl]
                    tim = tim_v[row, sl]
                    rre = rre_v[row, sl]
                    rim = rim_v[row, sl]
                    return acc + (rre * (hre * tre + him * tim)
                                  + rim * (hre * tim - him * tre))

                accc = lax.fori_loop(0, DIM // 16, ck,
                                     jnp.zeros((16,), jnp.float32), unroll=8)

                def ck2(k, acc):
                    sl = pl.ds(k * 16, 16)
                    hh = h_v[row, sl]
                    tt = t_v[row, sl]
                    rr = r_v[row, sl]
                    trr = tt * rr
                    t_v[row, sl] = trr
                    h_v[row, sl] = hh * rr
                    return acc + hh * trr

                accs = lax.fori_loop(0, D2 // 16, ck2,
                                     jnp.zeros((16,), jnp.float32), unroll=8)
                c_v[row] = accc
                stt_v[row] = accs
                return carry2

        lax.fori_loop(0, CH, row_body, 0)
        pltpu.sync_copy(t_v, u_hbm.at[pl.ds(rbase + off, CH)])
        pltpu.sync_copy(h_v, w_hbm.at[pl.ds(rbase + off, CH)])
        pltpu.sync_copy(r_v, r_hbm.at[pl.ds(rbase + off, CH)])
        pltpu.sync_copy(c_v, c_hbm.at[pl.ds(rbase + off, CH)])
        pltpu.sync_copy(stt_v, stt_hbm.at[pl.ds(rbase + off, CH)])
        return carry

    lax.fori_loop(0, nch, chunk_body, 0)

  return _sc_prep


def _score_block(bh_ref, bt_ref, k_ref,
                 visual_ref,
                 u_ref, w_ref, rm_ref, c_ref, stt_ref, tm_ref, wp_ref, bp_ref,
                 out_ref,
                 xh_buf, xt_buf, svis_ref, sem):
    i = pl.program_id(0)
    nb = pl.num_programs(0)
    kk = k_ref[0]

    def issue(block, slot):
        base = block * BM

        @pl.when(base + BM <= kk)
        def _():
            def body(j, carry):
                row = base + j
                pltpu.make_async_copy(
                    visual_ref.at[bh_ref[row]], xh_buf.at[slot, j],
                    sem.at[slot, 0]).start()
                pltpu.make_async_copy(
                    visual_ref.at[bt_ref[row]], xt_buf.at[slot, j],
                    sem.at[slot, 1]).start()
                return carry
            lax.fori_loop(0, BM, body, 0, unroll=8)

        @pl.when((base < kk) & (base + BM > kk))
        def _():
            def body(j, carry):
                row = base + j

                @pl.when(row < kk)
                def _():
                    pltpu.make_async_copy(
                        visual_ref.at[bh_ref[row]], xh_buf.at[slot, j],
                        sem.at[slot, 0]).start()
                    pltpu.make_async_copy(
                        visual_ref.at[bt_ref[row]], xt_buf.at[slot, j],
                        sem.at[slot, 1]).start()
                return carry
            lax.fori_loop(0, BM, body, 0, unroll=8)

    def wait(block, slot):
        base = block * BM

        @pl.when(base + BM <= kk)
        def _():
            pltpu.make_async_copy(
                visual_ref.at[pl.ds(0, BM)], xh_buf.at[slot],
                sem.at[slot, 0]).wait()
            pltpu.make_async_copy(
                visual_ref.at[pl.ds(0, BM)], xt_buf.at[slot],
                sem.at[slot, 1]).wait()

        @pl.when((base < kk) & (base + BM > kk))
        def _():
            def body(j, carry):
                row = base + j

                @pl.when(row < kk)
                def _():
                    pltpu.make_async_copy(
                        visual_ref.at[bh_ref[row]], xh_buf.at[slot, j],
                        sem.at[slot, 0]).wait()
                    pltpu.make_async_copy(
                        visual_ref.at[bt_ref[row]], xt_buf.at[slot, j],
                        sem.at[slot, 1]).wait()
                return carry
            lax.fori_loop(0, BM, body, 0, unroll=8)

    @pl.when(i == 0)
    def _():
        issue(0, 0)

    @pl.when(i + 1 < nb)
    def _():
        issue(i + 1, (i + 1) % 2)

    slot = i % 2
    wait(i, slot)

    @pl.when(i * BM < kk)
    def _():
        xh = xh_buf[slot]
        xt = xt_buf[slot]
        wp = wp_ref[...]
        bp = bp_ref[...]
        ha = lax.dot_general(xh, wp, (((1,), (1,)), ((), ())),
                             preferred_element_type=jnp.float32) + bp
        ta = lax.dot_general(xt, wp, (((1,), (1,)), ((), ())),
                             preferred_element_type=jnp.float32) + bp
        u = u_ref[...]
        w = w_ref[...]
        rm = rm_ref[...]
        svis_ref[0, ...] = jnp.sum(ha * u + ta * w, axis=-1)
        svis_ref[1, ...] = jnp.sum(ha * ta * rm, axis=-1)

    c = -jnp.sum(c_ref[0], axis=-1)
    stt = jnp.sum(stt_ref[0], axis=-1)
    tm = tm_ref[0, ...]
    s_hyb = svis_ref[0, ...]
    s_ii = svis_ref[1, ...]
    score = jnp.where(tm == 0, c - stt, 0.0)
    score = score + jnp.where(tm == 1, 2.0 * c - s_hyb, 0.0)
    score = score + jnp.where(tm == 2, c - s_ii, 0.0)
    out_ref[0, ...] = score


@functools.lru_cache(maxsize=None)
def _make_fused_score(bsz):
  nb = bsz // BM

  def _fused_score(bh, bt, karr, visual, u, w, rm, c, stt, tm, wp, bp):
    row2 = lambda: pl.BlockSpec((BM, D2), lambda i, *_: (i, 0))
    vec1 = lambda: pl.BlockSpec((1, 1, BM), lambda i, *_: (i, 0, 0))
    grid_spec = pltpu.PrefetchScalarGridSpec(
        num_scalar_prefetch=3,
        grid=(nb,),
        in_specs=[
            pl.BlockSpec(memory_space=pltpu.MemorySpace.HBM),
            row2(), row2(), row2(),
            pl.BlockSpec((1, BM, 16), lambda i, *_: (i, 0, 0)),
            pl.BlockSpec((1, BM, 16), lambda i, *_: (i, 0, 0)),
            vec1(),
            pl.BlockSpec((D2, VIS), lambda i, *_: (0, 0)),
            pl.BlockSpec((1, D2), lambda i, *_: (0, 0)),
        ],
        out_specs=pl.BlockSpec((1, 1, BM), lambda i, *_: (i, 0, 0)),
        scratch_shapes=[
            pltpu.VMEM((2, BM, VIS), jnp.float32),
            pltpu.VMEM((2, BM, VIS), jnp.float32),
            pltpu.VMEM((2, BM), jnp.float32),
            pltpu.SemaphoreType.DMA((2, 2)),
        ],
    )
    out = pl.pallas_call(
        _score_block,
        grid_spec=grid_spec,
        out_shape=jax.ShapeDtypeStruct((nb, 1, BM), jnp.float32),
    )(bh, bt, karr, visual, u, w, rm,
      c.reshape(nb, BM, 16), stt.reshape(nb, BM, 16), tm.reshape(nb, 1, BM),
      wp, bp.reshape(1, D2))
    return out.reshape(bsz)

  return _fused_score


NSPLIT = 8


def kernel(batch_h, batch_t, batch_r, task_mode, mode,
           ent_re, ent_im, ent_emb, rel_re, rel_im, rel_emb,
           visual, Wp, bp):
    hb = B // NSPLIT
    sc = _make_sc_prep(hb)
    fs = _make_fused_score(hb)
    karr = jnp.full((1,), hb, jnp.int32)
    parts = []
    for sp in range(NSPLIT):
        sl = slice(sp * hb, (sp + 1) * hb)
        c, stt, u, w, r = sc(batch_h[sl], batch_t[sl], batch_r[sl],
                             ent_re, ent_im, ent_emb,
                             rel_re, rel_im, rel_emb)
        parts.append(fs(batch_h[sl], batch_t[sl], karr, visual,
                        u, w, r, c, stt, task_mode[sl], Wp, bp))
    return jnp.concatenate(parts)


# speedup vs baseline: 1.0331x; 1.0331x over previous
import functools

import jax
import jax.numpy as jnp
from jax import lax
from jax.experimental import pallas as pl
from jax.experimental.pallas import tpu as pltpu
from jax.experimental.pallas import tpu_sc as plsc

B = 16384
DIM = 128
D2 = 2 * DIM
VIS = 4096
BM = 256
NB = B // BM

NW = 32
RPW = B // NW
CH = 64
NCH = RPW // CH

_sc_mesh = plsc.VectorSubcoreMesh(core_axis_name="c", subcore_axis_name="s")


@functools.lru_cache(maxsize=None)
def _make_sc_prep(bsz):
  rpw = bsz // NW
  nch = rpw // CH

  @functools.partial(
    pl.kernel,
    mesh=_sc_mesh,
    out_type=[
        jax.ShapeDtypeStruct((bsz, 16), jnp.float32),
        jax.ShapeDtypeStruct((bsz, 16), jnp.float32),
        jax.ShapeDtypeStruct((bsz, D2), jnp.float32),
        jax.ShapeDtypeStruct((bsz, D2), jnp.float32),
        jax.ShapeDtypeStruct((bsz, D2), jnp.float32),
    ],
    scratch_types=[
        pltpu.VMEM((rpw,), jnp.int32),
        pltpu.VMEM((rpw,), jnp.int32),
        pltpu.VMEM((rpw,), jnp.int32),
        pltpu.VMEM((CH, DIM), jnp.float32),
        pltpu.VMEM((CH, DIM), jnp.float32),
        pltpu.VMEM((CH, DIM), jnp.float32),
        pltpu.VMEM((CH, DIM), jnp.float32),
        pltpu.VMEM((CH, DIM), jnp.float32),
        pltpu.VMEM((CH, DIM), jnp.float32),
        pltpu.VMEM((CH, D2), jnp.float32),
        pltpu.VMEM((CH, D2), jnp.float32),
        pltpu.VMEM((CH, D2), jnp.float32),
        pltpu.VMEM((CH, 16), jnp.float32),
        pltpu.VMEM((CH, 16), jnp.float32),
        pltpu.SemaphoreType.DMA,
    ],
  )
  def _sc_prep(bh_hbm, bt_hbm, br_hbm,
               entre_hbm, entim_hbm, entemb_hbm,
               relre_hbm, relim_hbm, relemb_hbm,
               c_hbm, stt_hbm, u_hbm, w_hbm, r_hbm,
               bh_v, bt_v, br_v,
               hre_v, him_v, tre_v, tim_v, rre_v, rim_v,
               h_v, t_v, r_v, c_v, stt_v, sem):
    wid = lax.axis_index("s") * 2 + lax.axis_index("c")
    rbase = wid * rpw
    pltpu.sync_copy(bh_hbm.at[pl.ds(rbase, rpw)], bh_v)
    pltpu.sync_copy(bt_hbm.at[pl.ds(rbase, rpw)], bt_v)
    pltpu.sync_copy(br_hbm.at[pl.ds(rbase, rpw)], br_v)

    def chunk_body(ci, carry):
        off = ci * CH
        ih = bh_v.at[pl.ds(off, CH)]
        it = bt_v.at[pl.ds(off, CH)]
        ir = br_v.at[pl.ds(off, CH)]
        cps = [
            pltpu.make_async_copy(entre_hbm.at[ih], hre_v, sem),
            pltpu.make_async_copy(entim_hbm.at[ih], him_v, sem),
            pltpu.make_async_copy(entre_hbm.at[it], tre_v, sem),
            pltpu.make_async_copy(entim_hbm.at[it], tim_v, sem),
            pltpu.make_async_copy(relre_hbm.at[ir], rre_v, sem),
            pltpu.make_async_copy(relim_hbm.at[ir], rim_v, sem),
            pltpu.make_async_copy(entemb_hbm.at[ih], h_v, sem),
            pltpu.make_async_copy(entemb_hbm.at[it], t_v, sem),
            pltpu.make_async_copy(relemb_hbm.at[ir], r_v, sem),
        ]
        for cp in cps:
            cp.start()
        for cp in cps:
            cp.wait()

        def row_body(row, carry2):
                def ck(k, acc):
                    sl = pl.ds(k * 16, 16)
                    hre = hre_v[row, sl]
                    him = him_v[row, sl]
                    tre = tre_v[row, sl]
                    tim = tim_v[row, sl]
                    rre = rre_v[row, sl]
                    rim = rim_v[row, sl]
                    return acc + (rre * (hre * tre + him * tim)
                                  + rim * (hre * tim - him * tre))

                accc = lax.fori_loop(0, DIM // 16, ck,
                                     jnp.zeros((16,), jnp.float32), unroll=8)

                def ck2(k, acc):
                    sl = pl.ds(k * 16, 16)
                    hh = h_v[row, sl]
                    tt = t_v[row, sl]
                    rr = r_v[row, sl]
                    trr = tt * rr
                    t_v[row, sl] = trr
                    h_v[row, sl] = hh * rr
                    return acc + hh * trr

                accs = lax.fori_loop(0, D2 // 16, ck2,
                                     jnp.zeros((16,), jnp.float32), unroll=8)
                c_v[row] = accc
                stt_v[row] = accs
                return carry2

        lax.fori_loop(0, CH, row_body, 0)
        pltpu.sync_copy(t_v, u_hbm.at[pl.ds(rbase + off, CH)])
        pltpu.sync_copy(h_v, w_hbm.at[pl.ds(rbase + off, CH)])
        pltpu.sync_copy(r_v, r_hbm.at[pl.ds(rbase + off, CH)])
        pltpu.sync_copy(c_v, c_hbm.at[pl.ds(rbase + off, CH)])
        pltpu.sync_copy(stt_v, stt_hbm.at[pl.ds(rbase + off, CH)])
        return carry

    lax.fori_loop(0, nch, chunk_body, 0)

  return _sc_prep


def _score_block(bh_ref, bt_ref, k_ref,
                 visual_ref,
                 u_ref, w_ref, rm_ref, c_ref, stt_ref, tm_ref, wp_ref, bp_ref,
                 out_ref,
                 xh_buf, xt_buf, svis_ref, sem):
    i = pl.program_id(0)
    nb = pl.num_programs(0)
    kk = k_ref[0]

    def issue(block, slot):
        base = block * BM

        @pl.when(base + BM <= kk)
        def _():
            def body(j, carry):
                row = base + j
                pltpu.make_async_copy(
                    visual_ref.at[bh_ref[row]], xh_buf.at[slot, j],
                    sem.at[slot, 0]).start()
                pltpu.make_async_copy(
                    visual_ref.at[bt_ref[row]], xt_buf.at[slot, j],
                    sem.at[slot, 1]).start()
                return carry
            lax.fori_loop(0, BM, body, 0, unroll=16)

        @pl.when((base < kk) & (base + BM > kk))
        def _():
            def body(j, carry):
                row = base + j

                @pl.when(row < kk)
                def _():
                    pltpu.make_async_copy(
                        visual_ref.at[bh_ref[row]], xh_buf.at[slot, j],
                        sem.at[slot, 0]).start()
                    pltpu.make_async_copy(
                        visual_ref.at[bt_ref[row]], xt_buf.at[slot, j],
                        sem.at[slot, 1]).start()
                return carry
            lax.fori_loop(0, BM, body, 0, unroll=16)

    def wait(block, slot):
        base = block * BM

        @pl.when(base + BM <= kk)
        def _():
            pltpu.make_async_copy(
                visual_ref.at[pl.ds(0, BM)], xh_buf.at[slot],
                sem.at[slot, 0]).wait()
            pltpu.make_async_copy(
                visual_ref.at[pl.ds(0, BM)], xt_buf.at[slot],
                sem.at[slot, 1]).wait()

        @pl.when((base < kk) & (base + BM > kk))
        def _():
            def body(j, carry):
                row = base + j

                @pl.when(row < kk)
                def _():
                    pltpu.make_async_copy(
                        visual_ref.at[bh_ref[row]], xh_buf.at[slot, j],
                        sem.at[slot, 0]).wait()
                    pltpu.make_async_copy(
                        visual_ref.at[bt_ref[row]], xt_buf.at[slot, j],
                        sem.at[slot, 1]).wait()
                return carry
            lax.fori_loop(0, BM, body, 0, unroll=16)

    @pl.when(i == 0)
    def _():
        issue(0, 0)

    @pl.when(i + 1 < nb)
    def _():
        issue(i + 1, (i + 1) % 2)

    slot = i % 2
    wait(i, slot)

    @pl.when(i * BM < kk)
    def _():
        xh = xh_buf[slot]
        xt = xt_buf[slot]
        wp = wp_ref[...]
        bp = bp_ref[...]
        ha = lax.dot_general(xh, wp, (((1,), (1,)), ((), ())),
                             preferred_element_type=jnp.float32) + bp
        ta = lax.dot_general(xt, wp, (((1,), (1,)), ((), ())),
                             preferred_element_type=jnp.float32) + bp
        u = u_ref[...]
        w = w_ref[...]
        rm = rm_ref[...]
        svis_ref[0, ...] = jnp.sum(ha * u + ta * w, axis=-1)
        svis_ref[1, ...] = jnp.sum(ha * ta * rm, axis=-1)

    c = -jnp.sum(c_ref[0], axis=-1)
    stt = jnp.sum(stt_ref[0], axis=-1)
    tm = tm_ref[0, ...]
    s_hyb = svis_ref[0, ...]
    s_ii = svis_ref[1, ...]
    score = jnp.where(tm == 0, c - stt, 0.0)
    score = score + jnp.where(tm == 1, 2.0 * c - s_hyb, 0.0)
    score = score + jnp.where(tm == 2, c - s_ii, 0.0)
    out_ref[0, ...] = score


@functools.lru_cache(maxsize=None)
def _make_fused_score(bsz):
  nb = bsz // BM

  def _fused_score(bh, bt, karr, visual, u, w, rm, c, stt, tm, wp, bp):
    row2 = lambda: pl.BlockSpec((BM, D2), lambda i, *_: (i, 0))
    vec1 = lambda: pl.BlockSpec((1, 1, BM), lambda i, *_: (i, 0, 0))
    grid_spec = pltpu.PrefetchScalarGridSpec(
        num_scalar_prefetch=3,
        grid=(nb,),
        in_specs=[
            pl.BlockSpec(memory_space=pltpu.MemorySpace.HBM),
            row2(), row2(), row2(),
            pl.BlockSpec((1, BM, 16), lambda i, *_: (i, 0, 0)),
            pl.BlockSpec((1, BM, 16), lambda i, *_: (i, 0, 0)),
            vec1(),
            pl.BlockSpec((D2, VIS), lambda i, *_: (0, 0)),
            pl.BlockSpec((1, D2), lambda i, *_: (0, 0)),
        ],
        out_specs=pl.BlockSpec((1, 1, BM), lambda i, *_: (i, 0, 0)),
        scratch_shapes=[
            pltpu.VMEM((2, BM, VIS), jnp.float32),
            pltpu.VMEM((2, BM, VIS), jnp.float32),
            pltpu.VMEM((2, BM), jnp.float32),
            pltpu.SemaphoreType.DMA((2, 2)),
        ],
    )
    out = pl.pallas_call(
        _score_block,
        grid_spec=grid_spec,
        out_shape=jax.ShapeDtypeStruct((nb, 1, BM), jnp.float32),
    )(bh, bt, karr, visual, u, w, rm,
      c.reshape(nb, BM, 16), stt.reshape(nb, BM, 16), tm.reshape(nb, 1, BM),
      wp, bp.reshape(1, D2))
    return out.reshape(bsz)

  return _fused_score


NSPLIT = 4


def kernel(batch_h, batch_t, batch_r, task_mode, mode,
           ent_re, ent_im, ent_emb, rel_re, rel_im, rel_emb,
           visual, Wp, bp):
    hb = B // NSPLIT
    sc = _make_sc_prep(hb)
    fs = _make_fused_score(hb)
    karr = jnp.full((1,), hb, jnp.int32)
    parts = []
    for sp in range(NSPLIT):
        sl = slice(sp * hb, (sp + 1) * hb)
        c, stt, u, w, r = sc(batch_h[sl], batch_t[sl], batch_r[sl],
                             ent_re, ent_im, ent_emb,
                             rel_re, rel_im, rel_emb)
        parts.append(fs(batch_h[sl], batch_t[sl], karr, visual,
                        u, w, r, c, stt, task_mode[sl], Wp, bp))
    return jnp.concatenate(parts)


# speedup vs baseline: 1.0696x; 1.0353x over previous
import functools

import jax
import jax.numpy as jnp
from jax import lax
from jax.experimental import pallas as pl
from jax.experimental.pallas import tpu as pltpu
from jax.experimental.pallas import tpu_sc as plsc

B = 16384
DIM = 128
D2 = 2 * DIM
VIS = 4096
BM = 512
NB = B // BM

NW = 32
RPW = B // NW
CH = 64
NCH = RPW // CH

_sc_mesh = plsc.VectorSubcoreMesh(core_axis_name="c", subcore_axis_name="s")


@functools.lru_cache(maxsize=None)
def _make_sc_prep(bsz):
  rpw = bsz // NW
  nch = rpw // CH

  @functools.partial(
    pl.kernel,
    mesh=_sc_mesh,
    out_type=[
        jax.ShapeDtypeStruct((bsz, 16), jnp.float32),
        jax.ShapeDtypeStruct((bsz, 16), jnp.float32),
        jax.ShapeDtypeStruct((bsz, D2), jnp.float32),
        jax.ShapeDtypeStruct((bsz, D2), jnp.float32),
        jax.ShapeDtypeStruct((bsz, D2), jnp.float32),
    ],
    scratch_types=[
        pltpu.VMEM((rpw,), jnp.int32),
        pltpu.VMEM((rpw,), jnp.int32),
        pltpu.VMEM((rpw,), jnp.int32),
        pltpu.VMEM((CH, DIM), jnp.float32),
        pltpu.VMEM((CH, DIM), jnp.float32),
        pltpu.VMEM((CH, DIM), jnp.float32),
        pltpu.VMEM((CH, DIM), jnp.float32),
        pltpu.VMEM((CH, DIM), jnp.float32),
        pltpu.VMEM((CH, DIM), jnp.float32),
        pltpu.VMEM((CH, D2), jnp.float32),
        pltpu.VMEM((CH, D2), jnp.float32),
        pltpu.VMEM((CH, D2), jnp.float32),
        pltpu.VMEM((CH, 16), jnp.float32),
        pltpu.VMEM((CH, 16), jnp.float32),
        pltpu.SemaphoreType.DMA,
    ],
  )
  def _sc_prep(bh_hbm, bt_hbm, br_hbm,
               entre_hbm, entim_hbm, entemb_hbm,
               relre_hbm, relim_hbm, relemb_hbm,
               c_hbm, stt_hbm, u_hbm, w_hbm, r_hbm,
               bh_v, bt_v, br_v,
               hre_v, him_v, tre_v, tim_v, rre_v, rim_v,
               h_v, t_v, r_v, c_v, stt_v, sem):
    wid = lax.axis_index("s") * 2 + lax.axis_index("c")
    rbase = wid * rpw
    pltpu.sync_copy(bh_hbm.at[pl.ds(rbase, rpw)], bh_v)
    pltpu.sync_copy(bt_hbm.at[pl.ds(rbase, rpw)], bt_v)
    pltpu.sync_copy(br_hbm.at[pl.ds(rbase, rpw)], br_v)

    def chunk_body(ci, carry):
        off = ci * CH
        ih = bh_v.at[pl.ds(off, CH)]
        it = bt_v.at[pl.ds(off, CH)]
        ir = br_v.at[pl.ds(off, CH)]
        cps = [
            pltpu.make_async_copy(entre_hbm.at[ih], hre_v, sem),
            pltpu.make_async_copy(entim_hbm.at[ih], him_v, sem),
            pltpu.make_async_copy(entre_hbm.at[it], tre_v, sem),
            pltpu.make_async_copy(entim_hbm.at[it], tim_v, sem),
            pltpu.make_async_copy(relre_hbm.at[ir], rre_v, sem),
            pltpu.make_async_copy(relim_hbm.at[ir], rim_v, sem),
            pltpu.make_async_copy(entemb_hbm.at[ih], h_v, sem),
            pltpu.make_async_copy(entemb_hbm.at[it], t_v, sem),
            pltpu.make_async_copy(relemb_hbm.at[ir], r_v, sem),
        ]
        for cp in cps:
            cp.start()
        for cp in cps:
            cp.wait()

        def row_body(row, carry2):
                def ck(k, acc):
                    sl = pl.ds(k * 16, 16)
                    hre = hre_v[row, sl]
                    him = him_v[row, sl]
                    tre = tre_v[row, sl]
                    tim = tim_v[row, sl]
                    rre = rre_v[row, sl]
                    rim = rim_v[row, sl]
                    return acc + (rre * (hre * tre + him * tim)
                                  + rim * (hre * tim - him * tre))

                accc = lax.fori_loop(0, DIM // 16, ck,
                                     jnp.zeros((16,), jnp.float32), unroll=8)

                def ck2(k, acc):
                    sl = pl.ds(k * 16, 16)
                    hh = h_v[row, sl]
                    tt = t_v[row, sl]
                    rr = r_v[row, sl]
                    trr = tt * rr
                    t_v[row, sl] = trr
                    h_v[row, sl] = hh * rr
                    return acc + hh * trr

                accs = lax.fori_loop(0, D2 // 16, ck2,
                                     jnp.zeros((16,), jnp.float32), unroll=8)
                c_v[row] = accc
                stt_v[row] = accs
                return carry2

        lax.fori_loop(0, CH, row_body, 0)
        pltpu.sync_copy(t_v, u_hbm.at[pl.ds(rbase + off, CH)])
        pltpu.sync_copy(h_v, w_hbm.at[pl.ds(rbase + off, CH)])
        pltpu.sync_copy(r_v, r_hbm.at[pl.ds(rbase + off, CH)])
        pltpu.sync_copy(c_v, c_hbm.at[pl.ds(rbase + off, CH)])
        pltpu.sync_copy(stt_v, stt_hbm.at[pl.ds(rbase + off, CH)])
        return carry

    lax.fori_loop(0, nch, chunk_body, 0)

  return _sc_prep


def _score_block(bh_ref, bt_ref, k_ref,
                 visual_ref,
                 u_ref, w_ref, rm_ref, c_ref, stt_ref, tm_ref, wp_ref, bp_ref,
                 out_ref,
                 xh_buf, xt_buf, svis_ref, sem):
    i = pl.program_id(0)
    nb = pl.num_programs(0)
    kk = k_ref[0]

    def issue(block, slot):
        base = block * BM

        @pl.when(base + BM <= kk)
        def _():
            def body(j, carry):
                row = base + j
                pltpu.make_async_copy(
                    visual_ref.at[bh_ref[row]], xh_buf.at[slot, j],
                    sem.at[slot, 0]).start()
                pltpu.make_async_copy(
                    visual_ref.at[bt_ref[row]], xt_buf.at[slot, j],
                    sem.at[slot, 1]).start()
                return carry
            lax.fori_loop(0, BM, body, 0, unroll=16)

        @pl.when((base < kk) & (base + BM > kk))
        def _():
            def body(j, carry):
                row = base + j

                @pl.when(row < kk)
                def _():
                    pltpu.make_async_copy(
                        visual_ref.at[bh_ref[row]], xh_buf.at[slot, j],
                        sem.at[slot, 0]).start()
                    pltpu.make_async_copy(
                        visual_ref.at[bt_ref[row]], xt_buf.at[slot, j],
                        sem.at[slot, 1]).start()
                return carry
            lax.fori_loop(0, BM, body, 0, unroll=16)

    def wait(block, slot):
        base = block * BM

        @pl.when(base + BM <= kk)
        def _():
            pltpu.make_async_copy(
                visual_ref.at[pl.ds(0, BM)], xh_buf.at[slot],
                sem.at[slot, 0]).wait()
            pltpu.make_async_copy(
                visual_ref.at[pl.ds(0, BM)], xt_buf.at[slot],
                sem.at[slot, 1]).wait()

        @pl.when((base < kk) & (base + BM > kk))
        def _():
            def body(j, carry):
                row = base + j

                @pl.when(row < kk)
                def _():
                    pltpu.make_async_copy(
                        visual_ref.at[bh_ref[row]], xh_buf.at[slot, j],
                        sem.at[slot, 0]).wait()
                    pltpu.make_async_copy(
                        visual_ref.at[bt_ref[row]], xt_buf.at[slot, j],
                        sem.at[slot, 1]).wait()
                return carry
            lax.fori_loop(0, BM, body, 0, unroll=16)

    @pl.when(i == 0)
    def _():
        issue(0, 0)

    @pl.when(i + 1 < nb)
    def _():
        issue(i + 1, (i + 1) % 2)

    slot = i % 2
    wait(i, slot)

    @pl.when(i * BM < kk)
    def _():
        xh = xh_buf[slot]
        xt = xt_buf[slot]
        wp = wp_ref[...]
        bp = bp_ref[...]
        ha = lax.dot_general(xh, wp, (((1,), (1,)), ((), ())),
                             preferred_element_type=jnp.float32) + bp
        ta = lax.dot_general(xt, wp, (((1,), (1,)), ((), ())),
                             preferred_element_type=jnp.float32) + bp
        u = u_ref[...]
        w = w_ref[...]
        rm = rm_ref[...]
        svis_ref[0, ...] = jnp.sum(ha * u + ta * w, axis=-1)
        svis_ref[1, ...] = jnp.sum(ha * ta * rm, axis=-1)

    c = -jnp.sum(c_ref[0], axis=-1)
    stt = jnp.sum(stt_ref[0], axis=-1)
    tm = tm_ref[0, ...]
    s_hyb = svis_ref[0, ...]
    s_ii = svis_ref[1, ...]
    score = jnp.where(tm == 0, c - stt, 0.0)
    score = score + jnp.where(tm == 1, 2.0 * c - s_hyb, 0.0)
    score = score + jnp.where(tm == 2, c - s_ii, 0.0)
    out_ref[0, ...] = score


@functools.lru_cache(maxsize=None)
def _make_fused_score(bsz):
  nb = bsz // BM

  def _fused_score(bh, bt, karr, visual, u, w, rm, c, stt, tm, wp, bp):
    row2 = lambda: pl.BlockSpec((BM, D2), lambda i, *_: (i, 0))
    vec1 = lambda: pl.BlockSpec((1, 1, BM), lambda i, *_: (i, 0, 0))
    grid_spec = pltpu.PrefetchScalarGridSpec(
        num_scalar_prefetch=3,
        grid=(nb,),
        in_specs=[
            pl.BlockSpec(memory_space=pltpu.MemorySpace.HBM),
            row2(), row2(), row2(),
            pl.BlockSpec((1, BM, 16), lambda i, *_: (i, 0, 0)),
            pl.BlockSpec((1, BM, 16), lambda i, *_: (i, 0, 0)),
            vec1(),
            pl.BlockSpec((D2, VIS), lambda i, *_: (0, 0)),
            pl.BlockSpec((1, D2), lambda i, *_: (0, 0)),
        ],
        out_specs=pl.BlockSpec((1, 1, BM), lambda i, *_: (i, 0, 0)),
        scratch_shapes=[
            pltpu.VMEM((2, BM, VIS), jnp.float32),
            pltpu.VMEM((2, BM, VIS), jnp.float32),
            pltpu.VMEM((2, BM), jnp.float32),
            pltpu.SemaphoreType.DMA((2, 2)),
        ],
    )
    out = pl.pallas_call(
        _score_block,
        grid_spec=grid_spec,
        out_shape=jax.ShapeDtypeStruct((nb, 1, BM), jnp.float32),
    )(bh, bt, karr, visual, u, w, rm,
      c.reshape(nb, BM, 16), stt.reshape(nb, BM, 16), tm.reshape(nb, 1, BM),
      wp, bp.reshape(1, D2))
    return out.reshape(bsz)

  return _fused_score


NSPLIT = 4


def kernel(batch_h, batch_t, batch_r, task_mode, mode,
           ent_re, ent_im, ent_emb, rel_re, rel_im, rel_emb,
           visual, Wp, bp):
    hb = B // NSPLIT
    sc = _make_sc_prep(hb)
    fs = _make_fused_score(hb)
    karr = jnp.full((1,), hb, jnp.int32)
    parts = []
    for sp in range(NSPLIT):
        sl = slice(sp * hb, (sp + 1) * hb)
        c, stt, u, w, r = sc(batch_h[sl], batch_t[sl], batch_r[sl],
                             ent_re, ent_im, ent_emb,
                             rel_re, rel_im, rel_emb)
        parts.append(fs(batch_h[sl], batch_t[sl], karr, visual,
                        u, w, r, c, stt, task_mode[sl], Wp, bp))
    return jnp.concatenate(parts)
